# baseline (speedup 1.0000x reference)
import functools

import jax
import jax.numpy as jnp
from jax import lax
from jax.experimental import pallas as pl
from jax.experimental.pallas import tpu as pltpu
from jax.experimental.pallas import tpu_sc as plsc

_NWORKERS = 32
_CHUNK = 32
_BT = 1024
_LN_EPS = 1e-12


@functools.lru_cache(maxsize=None)
def _build_gather(n_tok, hidden):
    spw = n_tok // _NWORKERS
    n_pairs = spw // (2 * _CHUNK)
    mesh = plsc.VectorSubcoreMesh(core_axis_name="c", subcore_axis_name="s")
    buf_t = pltpu.VMEM((_CHUNK, hidden), jnp.float32)

    @functools.partial(
        pl.kernel,
        out_type=jax.ShapeDtypeStruct((n_tok, hidden), jnp.float32),
        mesh=mesh,
        compiler_params=pltpu.CompilerParams(needs_layout_passes=False),
        scratch_types=[
            pltpu.VMEM((spw,), jnp.int32),
            buf_t, buf_t,
            pltpu.SemaphoreType.DMA,
            pltpu.SemaphoreType.DMA,
        ],
    )
    def gather_kernel(ids_hbm, wemb, out_hbm, ids_v, wa, wb, sem_g, sem_o):
        wid = lax.axis_index("s") * mesh.num_cores + lax.axis_index("c")
        base = wid * spw
        pltpu.sync_copy(ids_hbm.at[pl.ds(base, spw)], ids_v)

        def issue(c, w):
            off = pl.multiple_of(c * _CHUNK, _CHUNK)
            for h in range(_CHUNK // 16):
                pltpu.async_copy(wemb.at[ids_v[pl.ds(off + h * 16, 16)]],
                                 w.at[pl.ds(h * 16, 16)], sem_g)

        def wait_gather(w):
            pltpu.make_async_copy(wemb.at[pl.ds(0, _CHUNK)], w, sem_g).wait()

        def writeback(c, w):
            off = pl.multiple_of(c * _CHUNK, _CHUNK)
            pltpu.async_copy(w, out_hbm.at[pl.ds(base + off, _CHUNK)], sem_o)

        def drain_out(w):
            pltpu.make_async_copy(wemb.at[pl.ds(0, _CHUNK)], w, sem_o).wait()

        issue(0, wa)

        def pair_body(cp, carry):
            c0 = cp * 2
            wait_gather(wa)

            @pl.when(cp > 0)
            def _():
                drain_out(wb)
            issue(c0 + 1, wb)
            writeback(c0, wa)
            wait_gather(wb)

            @pl.when(cp + 1 < n_pairs)
            def _():
                drain_out(wa)
                issue(c0 + 2, wa)
            writeback(c0 + 1, wb)
            return carry

        lax.fori_loop(0, n_pairs, pair_body, 0)
        drain_out(wa)
        drain_out(wb)

    return gather_kernel


def _ln_body(wrows_ref, pos_ref, ttf_ref, kkf_ref, temb_ref, kemb_ref,
             gamma_ref, beta_ref, out_ref):
    ttf = ttf_ref[...]
    kkf = kkf_ref[...]
    t0 = temb_ref[0, :]
    t1 = temb_ref[1, :]
    k0 = kemb_ref[0, :]
    k1 = kemb_ref[1, :]
    k2 = kemb_ref[2, :]
    base = wrows_ref[...] + pos_ref[...] + (t0 + k0)[None, :]
    v = (base
         + ttf * (t1 - t0)[None, :]
         + kkf * (k1 - k0)[None, :]
         + (kkf * (kkf - 1.0) * 0.5) * (k2 - 2.0 * k1 + k0)[None, :])
    mean = jnp.mean(v, axis=-1, keepdims=True)
    cv = v - mean
    var = jnp.mean(cv * cv, axis=-1, keepdims=True)
    rstd = lax.rsqrt(var + _LN_EPS)
    out_ref[...] = cv * rstd * gamma_ref[...] + beta_ref[...]


@functools.lru_cache(maxsize=None)
def _build_ln(n_tok, seq_len, hidden):
    s_blocks = seq_len // _BT
    n_batch = n_tok // seq_len

    def tok(si, bi):
        return (bi * s_blocks + si, 0)

    return pl.pallas_call(
        _ln_body,
        grid=(s_blocks, n_batch),
        in_specs=[
            pl.BlockSpec((_BT, hidden), tok),
            pl.BlockSpec((_BT, hidden), lambda si, bi: (si, 0)),
            pl.BlockSpec((_BT, 1), tok),
            pl.BlockSpec((_BT, 1), tok),
            pl.BlockSpec((2, hidden), lambda si, bi: (0, 0)),
            pl.BlockSpec((3, hidden), lambda si, bi: (0, 0)),
            pl.BlockSpec((1, hidden), lambda si, bi: (0, 0)),
            pl.BlockSpec((1, hidden), lambda si, bi: (0, 0)),
        ],
        out_specs=pl.BlockSpec((_BT, hidden), tok),
        out_shape=jax.ShapeDtypeStruct((n_tok, hidden), jnp.float32),
    )


def kernel(input_ids, token_type_ids, task_type_ids, word_emb, pos_emb,
           type_emb, task_emb, ln_gamma, ln_beta):
    b, s = input_ids.shape
    hidden = word_emb.shape[1]
    n_tok = b * s
    ids_flat = input_ids.reshape(b, s).astype(jnp.int32)
    ttf = token_type_ids.reshape(b, s, 1).astype(jnp.float32)
    kkf = task_type_ids.reshape(b, s, 1).astype(jnp.float32)
    gather_fn = _build_gather(s, hidden)
    ln_fn = _build_ln(s, s, hidden)
    wrows = [gather_fn(ids_flat[i], word_emb) for i in range(b)]
    outs = [ln_fn(wrows[i], pos_emb, ttf[i], kkf[i], type_emb, task_emb,
                  ln_gamma.reshape(1, -1), ln_beta.reshape(1, -1))
            for i in range(b)]
    return jnp.stack(outs).reshape(b, s, hidden)


# speedup vs baseline: 1.3680x; 1.3680x over previous
import functools

import jax
import jax.numpy as jnp
from jax import lax
from jax.experimental import pallas as pl
from jax.experimental.pallas import tpu as pltpu
from jax.experimental.pallas import tpu_sc as plsc

_NWORKERS = 32
_CHUNK = 32
_BT = 512
_LN_EPS = 1e-12


@functools.lru_cache(maxsize=None)
def _build_gather(n_tok, hidden):
    spw = n_tok // _NWORKERS
    n_pairs = spw // (2 * _CHUNK)
    mesh = plsc.VectorSubcoreMesh(core_axis_name="c", subcore_axis_name="s")
    buf_t = pltpu.VMEM((_CHUNK, hidden), jnp.float32)

    @functools.partial(
        pl.kernel,
        out_type=jax.ShapeDtypeStruct((n_tok, hidden), jnp.float32),
        mesh=mesh,
        compiler_params=pltpu.CompilerParams(needs_layout_passes=False),
        scratch_types=[
            pltpu.VMEM((spw,), jnp.int32),
            buf_t, buf_t,
            pltpu.SemaphoreType.DMA,
            pltpu.SemaphoreType.DMA,
        ],
    )
    def gather_kernel(ids_hbm, wemb, out_hbm, ids_v, wa, wb, sem_g, sem_o):
        wid = lax.axis_index("s") * mesh.num_cores + lax.axis_index("c")
        base = wid * spw
        pltpu.sync_copy(ids_hbm.at[pl.ds(base, spw)], ids_v)

        def issue(c, w):
            off = pl.multiple_of(c * _CHUNK, _CHUNK)
            for h in range(_CHUNK // 16):
                pltpu.async_copy(wemb.at[ids_v[pl.ds(off + h * 16, 16)]],
                                 w.at[pl.ds(h * 16, 16)], sem_g)

        def wait_gather(w):
            pltpu.make_async_copy(wemb.at[pl.ds(0, _CHUNK)], w, sem_g).wait()

        def writeback(c, w):
            off = pl.multiple_of(c * _CHUNK, _CHUNK)
            pltpu.async_copy(w, out_hbm.at[pl.ds(base + off, _CHUNK)], sem_o)

        def drain_out(w):
            pltpu.make_async_copy(wemb.at[pl.ds(0, _CHUNK)], w, sem_o).wait()

        issue(0, wa)

        def pair_body(cp, carry):
            c0 = cp * 2
            wait_gather(wa)

            @pl.when(cp > 0)
            def _():
                drain_out(wb)
            issue(c0 + 1, wb)
            writeback(c0, wa)
            wait_gather(wb)

            @pl.when(cp + 1 < n_pairs)
            def _():
                drain_out(wa)
                issue(c0 + 2, wa)
            writeback(c0 + 1, wb)
            return carry

        lax.fori_loop(0, n_pairs, pair_body, 0)
        drain_out(wa)
        drain_out(wb)

    return gather_kernel


def _ln_body(wrows_ref, pos_ref, ttf_ref, kkf_ref, temb_ref, kemb_ref,
             gamma_ref, beta_ref, out_ref):
    ttf = ttf_ref[...]
    kkf = kkf_ref[...]
    t0 = temb_ref[0, :]
    t1 = temb_ref[1, :]
    k0 = kemb_ref[0, :]
    k1 = kemb_ref[1, :]
    k2 = kemb_ref[2, :]
    base = wrows_ref[...] + pos_ref[...] + (t0 + k0)[None, :]
    v = (base
         + ttf * (t1 - t0)[None, :]
         + kkf * (k1 - k0)[None, :]
         + (kkf * (kkf - 1.0) * 0.5) * (k2 - 2.0 * k1 + k0)[None, :])
    mean = jnp.mean(v, axis=-1, keepdims=True)
    cv = v - mean
    var = jnp.mean(cv * cv, axis=-1, keepdims=True)
    rstd = lax.rsqrt(var + _LN_EPS)
    out_ref[...] = cv * rstd * gamma_ref[...] + beta_ref[...]


@functools.lru_cache(maxsize=None)
def _build_ln(n_tok, seq_len, hidden):
    s_blocks = seq_len // _BT
    n_batch = n_tok // seq_len

    def tok(si, bi):
        return (bi * s_blocks + si, 0)

    return pl.pallas_call(
        _ln_body,
        grid=(s_blocks, n_batch),
        in_specs=[
            pl.BlockSpec((_BT, hidden), tok),
            pl.BlockSpec((_BT, hidden), lambda si, bi: (si, 0)),
            pl.BlockSpec((_BT, 1), tok),
            pl.BlockSpec((_BT, 1), tok),
            pl.BlockSpec((2, hidden), lambda si, bi: (0, 0)),
            pl.BlockSpec((3, hidden), lambda si, bi: (0, 0)),
            pl.BlockSpec((1, hidden), lambda si, bi: (0, 0)),
            pl.BlockSpec((1, hidden), lambda si, bi: (0, 0)),
        ],
        out_specs=pl.BlockSpec((_BT, hidden), tok),
        out_shape=jax.ShapeDtypeStruct((n_tok, hidden), jnp.float32),
    )


def kernel(input_ids, token_type_ids, task_type_ids, word_emb, pos_emb,
           type_emb, task_emb, ln_gamma, ln_beta):
    b, s = input_ids.shape
    hidden = word_emb.shape[1]
    n_tok = b * s
    wrows = _build_gather(n_tok, hidden)(
        input_ids.reshape(-1).astype(jnp.int32), word_emb)
    out = _build_ln(n_tok, s, hidden)(
        wrows,
        pos_emb,
        token_type_ids.reshape(-1, 1).astype(jnp.float32),
        task_type_ids.reshape(-1, 1).astype(jnp.float32),
        type_emb,
        task_emb,
        ln_gamma.reshape(1, -1),
        ln_beta.reshape(1, -1),
    )
    return out.reshape(b, s, hidden)


# speedup vs baseline: 1.4517x; 1.0612x over previous
import functools

import jax
import jax.numpy as jnp
from jax import lax
from jax.experimental import pallas as pl
from jax.experimental.pallas import tpu as pltpu
from jax.experimental.pallas import tpu_sc as plsc

_NWORKERS = 32
_CHUNK = 64
_BT = 1024
_LN_EPS = 1e-12


@functools.lru_cache(maxsize=None)
def _build_gather(n_tok, hidden):
    spw = n_tok // _NWORKERS
    n_pairs = spw // (2 * _CHUNK)
    mesh = plsc.VectorSubcoreMesh(core_axis_name="c", subcore_axis_name="s")
    buf_t = pltpu.VMEM((_CHUNK, hidden), jnp.float32)

    @functools.partial(
        pl.kernel,
        out_type=jax.ShapeDtypeStruct((n_tok, hidden), jnp.float32),
        mesh=mesh,
        compiler_params=pltpu.CompilerParams(needs_layout_passes=False),
        scratch_types=[
            pltpu.VMEM((spw,), jnp.int32),
            buf_t, buf_t,
            pltpu.SemaphoreType.DMA,
            pltpu.SemaphoreType.DMA,
        ],
    )
    def gather_kernel(ids_hbm, wemb, out_hbm, ids_v, wa, wb, sem_g, sem_o):
        wid = lax.axis_index("s") * mesh.num_cores + lax.axis_index("c")
        base = wid * spw
        pltpu.sync_copy(ids_hbm.at[pl.ds(base, spw)], ids_v)

        def issue(c, w):
            off = pl.multiple_of(c * _CHUNK, _CHUNK)
            for h in range(_CHUNK // 16):
                pltpu.async_copy(wemb.at[ids_v[pl.ds(off + h * 16, 16)]],
                                 w.at[pl.ds(h * 16, 16)], sem_g)

        def wait_gather(w):
            pltpu.make_async_copy(wemb.at[pl.ds(0, _CHUNK)], w, sem_g).wait()

        def writeback(c, w):
            off = pl.multiple_of(c * _CHUNK, _CHUNK)
            pltpu.async_copy(w, out_hbm.at[pl.ds(base + off, _CHUNK)], sem_o)

        def drain_out(w):
            pltpu.make_async_copy(wemb.at[pl.ds(0, _CHUNK)], w, sem_o).wait()

        issue(0, wa)

        def pair_body(cp, carry):
            c0 = cp * 2
            wait_gather(wa)

            @pl.when(cp > 0)
            def _():
                drain_out(wb)
            issue(c0 + 1, wb)
            writeback(c0, wa)
            wait_gather(wb)

            @pl.when(cp + 1 < n_pairs)
            def _():
                drain_out(wa)
                issue(c0 + 2, wa)
            writeback(c0 + 1, wb)
            return carry

        lax.fori_loop(0, n_pairs, pair_body, 0)
        drain_out(wa)
        drain_out(wb)

    return gather_kernel


def _ln_body(wrows_ref, pos_ref, ttf_ref, kkf_ref, temb_ref, kemb_ref,
             gamma_ref, beta_ref, out_ref):
    ttf = ttf_ref[...]
    kkf = kkf_ref[...]
    t0 = temb_ref[0, :]
    t1 = temb_ref[1, :]
    k0 = kemb_ref[0, :]
    k1 = kemb_ref[1, :]
    k2 = kemb_ref[2, :]
    base = wrows_ref[...] + pos_ref[...] + (t0 + k0)[None, :]
    v = (base
         + ttf * (t1 - t0)[None, :]
         + kkf * (k1 - k0)[None, :]
         + (kkf * (kkf - 1.0) * 0.5) * (k2 - 2.0 * k1 + k0)[None, :])
    mean = jnp.mean(v, axis=-1, keepdims=True)
    cv = v - mean
    var = jnp.mean(cv * cv, axis=-1, keepdims=True)
    rstd = lax.rsqrt(var + _LN_EPS)
    out_ref[...] = cv * rstd * gamma_ref[...] + beta_ref[...]


@functools.lru_cache(maxsize=None)
def _build_ln(n_tok, seq_len, hidden):
    s_blocks = seq_len // _BT
    n_batch = n_tok // seq_len

    def tok(si, bi):
        return (bi * s_blocks + si, 0)

    return pl.pallas_call(
        _ln_body,
        grid=(s_blocks, n_batch),
        in_specs=[
            pl.BlockSpec((_BT, hidden), tok),
            pl.BlockSpec((_BT, hidden), lambda si, bi: (si, 0)),
            pl.BlockSpec((_BT, 1), tok),
            pl.BlockSpec((_BT, 1), tok),
            pl.BlockSpec((2, hidden), lambda si, bi: (0, 0)),
            pl.BlockSpec((3, hidden), lambda si, bi: (0, 0)),
            pl.BlockSpec((1, hidden), lambda si, bi: (0, 0)),
            pl.BlockSpec((1, hidden), lambda si, bi: (0, 0)),
        ],
        out_specs=pl.BlockSpec((_BT, hidden), tok),
        out_shape=jax.ShapeDtypeStruct((n_tok, hidden), jnp.float32),
    )


def kernel(input_ids, token_type_ids, task_type_ids, word_emb, pos_emb,
           type_emb, task_emb, ln_gamma, ln_beta):
    b, s = input_ids.shape
    hidden = word_emb.shape[1]
    n_tok = b * s
    wrows = _build_gather(n_tok, hidden)(
        input_ids.reshape(-1).astype(jnp.int32), word_emb)
    out = _build_ln(n_tok, s, hidden)(
        wrows,
        pos_emb,
        token_type_ids.reshape(-1, 1).astype(jnp.float32),
        task_type_ids.reshape(-1, 1).astype(jnp.float32),
        type_emb,
        task_emb,
        ln_gamma.reshape(1, -1),
        ln_beta.reshape(1, -1),
    )
    return out.reshape(b, s, hidden)
